# baseline (speedup 1.0000x reference)
import jax
import jax.numpy as jnp
from jax import lax
from jax.experimental import pallas as pl
from jax.experimental.pallas import tpu as pltpu
from jax.experimental.pallas import tpu_sc as plsc

B = 64
N = 4096
KMAX = 64
ONE_BITS = 0x3F800000
BI = 8
L = 16
ROWS_PER_TILE = 2


def _sc_rowmax_kernel(w_hbm, mx_hbm, wrow, mx_v):
    wid = lax.axis_index("s") * 2 + lax.axis_index("c")
    base = wid * ROWS_PER_TILE

    for r in range(ROWS_PER_TILE):
        row = base + r
        pltpu.sync_copy(w_hbm.at[row], wrow)

        def mx_step(i, m):
            return jnp.maximum(m, wrow[pl.ds(i * L, L)])

        m = lax.fori_loop(0, N // L, mx_step, jnp.zeros((L,), jnp.float32))
        s = m[0]
        for j in range(1, L):
            s = jnp.maximum(s, m[j])
        mx_v[...] = jnp.full((L,), s, jnp.float32)
        pltpu.sync_copy(mx_v, mx_hbm.at[row])


def _expand_kernel(w_ref, mx_ref, o_ref, t_s):
    g = pl.program_id(0)
    w = w_ref[...]

    @pl.when(g == 0)
    def _():
        ent = -(w * jnp.log(w + 1e-08)).sum(axis=-1, keepdims=True)
        k = jnp.clip((KMAX * (1.0 - ent)).astype(jnp.int32), 1, KMAX)
        kf = k.astype(jnp.float32)

        mxb = lax.bitcast_convert_type(mx_ref[...][:, :1], jnp.int32)
        fast = k == 1
        lo = jnp.where(fast, mxb, 0)
        hi = jnp.where(fast, mxb + 1, ONE_BITS)

        def not_done(carry):
            lo, hi = carry
            return jnp.any(hi - lo > 1)

        def step(carry):
            lo, hi = carry
            mid = (lo + hi) >> 1
            cand = lax.bitcast_convert_type(mid, jnp.float32)
            cnt = jnp.where(w >= cand, 1.0, 0.0).sum(axis=-1, keepdims=True)
            ge = cnt >= kf
            lo = jnp.where(ge, mid, lo)
            hi = jnp.where(ge, hi, mid)
            return lo, hi

        lo, hi = lax.while_loop(not_done, step, (lo, hi))
        t_s[...] = lax.bitcast_convert_type(lo, jnp.float32)

    t = t_s[pl.ds(g * BI, BI), :]
    wb = w[None, :, :]
    num = jnp.where(wb >= t[:, :, None], wb, 0.0)
    s = num.sum(axis=-1, keepdims=True)
    o_ref[...] = num * (1.0 / (s + 1e-08))


@jax.jit
def kernel(weights):
    sc_rowmax = pl.kernel(
        _sc_rowmax_kernel,
        out_type=jax.ShapeDtypeStruct((B, L), jnp.float32),
        mesh=plsc.VectorSubcoreMesh(core_axis_name="c", subcore_axis_name="s"),
        scratch_types=[
            pltpu.VMEM((N,), jnp.float32),
            pltpu.VMEM((L,), jnp.float32),
        ],
    )
    mx = sc_rowmax(weights)

    out = pl.pallas_call(
        _expand_kernel,
        grid=(B // BI,),
        in_specs=[
            pl.BlockSpec((B, N), lambda g: (0, 0)),
            pl.BlockSpec((B, L), lambda g: (0, 0)),
        ],
        out_specs=pl.BlockSpec((BI, B, N), lambda g: (g, 0, 0)),
        out_shape=jax.ShapeDtypeStruct((B, B, N), jnp.float32),
        scratch_shapes=[pltpu.VMEM((B, 1), jnp.float32)],
    )(weights, mx)
    return out


# speedup vs baseline: 1.9640x; 1.9640x over previous
import jax
import jax.numpy as jnp
from jax import lax
from jax.experimental import pallas as pl
from jax.experimental.pallas import tpu as pltpu
from jax.experimental.pallas import tpu_sc as plsc

B = 64
N = 4096
KMAX = 64
ONE_BITS = 0x3F800000
BI = 8
L = 16
ROWS_PER_TILE = 2


def _sc_rowmax_kernel(w_hbm, mx_hbm, wrow, mx_v):
    wid = lax.axis_index("s") * 2 + lax.axis_index("c")
    base = wid * ROWS_PER_TILE

    for r in range(ROWS_PER_TILE):
        row = base + r
        pltpu.sync_copy(w_hbm.at[row], wrow)

        def mx_step(i, m):
            return jnp.maximum(m, wrow[pl.ds(i * L, L)])

        m = lax.fori_loop(0, N // L, mx_step, jnp.zeros((L,), jnp.float32))
        s = m[0]
        for j in range(1, L):
            s = jnp.maximum(s, m[j])
        mx_v[...] = jnp.full((L,), s, jnp.float32)
        pltpu.sync_copy(mx_v, mx_hbm.at[row])


def _expand_kernel(w_ref, o_ref, t_s):
    g = pl.program_id(0)
    w = w_ref[...]

    @pl.when(g == 0)
    def _():
        ent = -(w * jnp.log(w + 1e-08)).sum(axis=-1, keepdims=True)
        k = jnp.clip((KMAX * (1.0 - ent)).astype(jnp.int32), 1, KMAX)
        kf = k.astype(jnp.float32)

        mx = w.max(axis=-1, keepdims=True)
        mxb = lax.bitcast_convert_type(mx, jnp.int32)
        fast = k == 1
        lo = jnp.where(fast, mxb, 0)
        hi = jnp.where(fast, mxb + 1, ONE_BITS)

        def not_done(carry):
            lo, hi = carry
            return jnp.any(hi - lo > 1)

        def step(carry):
            lo, hi = carry
            mid = (lo + hi) >> 1
            cand = lax.bitcast_convert_type(mid, jnp.float32)
            cnt = jnp.where(w >= cand, 1.0, 0.0).sum(axis=-1, keepdims=True)
            ge = cnt >= kf
            lo = jnp.where(ge, mid, lo)
            hi = jnp.where(ge, hi, mid)
            return lo, hi

        lo, hi = lax.while_loop(not_done, step, (lo, hi))
        t_s[...] = lax.bitcast_convert_type(lo, jnp.float32)

    t = t_s[pl.ds(g * BI, BI), :]
    wb = w[None, :, :]
    num = jnp.where(wb >= t[:, :, None], wb, 0.0)
    s = num.sum(axis=-1, keepdims=True)
    o_ref[...] = num * (1.0 / (s + 1e-08))


@jax.jit
def kernel(weights):
    out = pl.pallas_call(
        _expand_kernel,
        grid=(B // BI,),
        in_specs=[
            pl.BlockSpec((B, N), lambda g: (0, 0)),
        ],
        out_specs=pl.BlockSpec((BI, B, N), lambda g: (g, 0, 0)),
        out_shape=jax.ShapeDtypeStruct((B, B, N), jnp.float32),
        scratch_shapes=[pltpu.VMEM((B, 1), jnp.float32)],
    )(weights)
    return out
